# baseline (speedup 1.0000x reference)
import functools
import math

import jax
import jax.numpy as jnp
from jax import lax
from jax.experimental import pallas as pl
from jax.experimental.pallas import tpu as pltpu
from jax.experimental.pallas import tpu_sc as plsc

BINSIZE = 200
WIDTH = 20000
BINWIDTH = 100
PADW = 128
N_REGIONS = 50000
N_CLUSTERS = 8
N_FRAG = 1000000
N_CELLS = 4096
N_REGIONS_OI = 1024

NC, NS = 2, 16
NW = NC * NS

TILE = 2000
NT = N_FRAG // TILE
VECS = TILE // 16

HEIGHTS_PAD = N_REGIONS_OI * N_CLUSTERS * PADW
HS = HEIGHTS_PAD // NS

_mesh = plsc.VectorSubcoreMesh(core_axis_name="c", subcore_axis_name="s")


_G = 16


def _heights_body(s_ref, *refs):
    in_refs = refs[:2 * _G]
    o_ref = refs[2 * _G]
    for g in range(_G):
        b = in_refs[g][...]
        d = in_refs[_G + g][...]
        u = b + d
        m = jnp.max(u, axis=-1, keepdims=True)
        lse = jnp.log(jnp.sum(jnp.exp(u - m), axis=-1, keepdims=True)) + m
        h = u - lse - math.log(BINSIZE)
        o_ref[g, :, :BINWIDTH] = h[0]


def _heights(baseline_weight, delta_logit_weight, regions_oi):
    def b_spec(g):
        return pl.BlockSpec((1, 1, BINWIDTH),
                            lambda i, s, g=g: (s[_G * i + g], 0, 0))

    def d_spec(g):
        return pl.BlockSpec((1, N_CLUSTERS, BINWIDTH),
                            lambda i, s, g=g: (s[_G * i + g], 0, 0))

    grid_spec = pltpu.PrefetchScalarGridSpec(
        num_scalar_prefetch=1,
        grid=(N_REGIONS_OI // _G,),
        in_specs=[b_spec(g) for g in range(_G)]
                 + [d_spec(g) for g in range(_G)],
        out_specs=pl.BlockSpec((_G, N_CLUSTERS, PADW),
                               lambda i, s: (i, 0, 0)),
    )
    return pl.pallas_call(
        _heights_body,
        grid_spec=grid_spec,
        out_shape=jax.ShapeDtypeStruct((N_REGIONS_OI, N_CLUSTERS, PADW),
                                       jnp.float32),
    )(regions_oi, *([baseline_weight.reshape(N_REGIONS, 1, BINWIDTH)] * _G),
      *([delta_logit_weight] * _G))


@functools.partial(
    pl.kernel,
    out_type=jax.ShapeDtypeStruct((2 * N_FRAG,), jnp.float32),
    mesh=_mesh,
    compiler_params=pltpu.CompilerParams(needs_layout_passes=False),
    scratch_types=[
        pltpu.VMEM((N_CELLS,), jnp.int32),
        pltpu.VMEM((16,), jnp.float32),
        pltpu.VMEM((16,), jnp.float32),
        pltpu.VMEM((2 * TILE,), jnp.int32),
        pltpu.VMEM((TILE,), jnp.int32),
        pltpu.VMEM((TILE,), jnp.int32),
        pltpu.VMEM((TILE,), jnp.int32),
        pltpu.VMEM((TILE,), jnp.float32),
        pltpu.VMEM((2 * TILE,), jnp.float32),
        pltpu.VMEM_SHARED((HEIGHTS_PAD,), jnp.float32),
        pltpu.SemaphoreType.DMA,
    ],
)
def _frag_phase(h_hbm, c01_hbm, reg_hbm, cell_hbm, labels_hbm, cin_hbm,
                cout_hbm, out_hbm,
                labels_v, cin_v, cout_v, c01_v, reg_v, cell_v, idx_v,
                gath_v, out_v, h_sp, sem):
    sid = lax.axis_index("s")
    cid = lax.axis_index("c")
    wid = sid * NC + cid

    pltpu.sync_copy(h_hbm.at[pl.ds(sid * HS, HS)], h_sp.at[pl.ds(sid * HS, HS)])
    pltpu.sync_copy(labels_hbm, labels_v)
    pltpu.sync_copy(cin_hbm, cin_v)
    pltpu.sync_copy(cout_hbm, cout_v)
    plsc.subcore_barrier()

    lanes = lax.iota(jnp.int32, 16)
    cin = cin_v[...]
    cout = cout_v[...]

    n_tiles = (NT - wid + NW - 1) // NW

    def tile_body(i, carry):
        t = wid + i * NW
        base = t * TILE
        pltpu.sync_copy(c01_hbm.at[pl.ds(2 * base, 2 * TILE)], c01_v)
        pltpu.sync_copy(reg_hbm.at[pl.ds(base, TILE)], reg_v)
        pltpu.sync_copy(cell_hbm.at[pl.ds(base, TILE)], cell_v)

        def vec_body(j, c):
            o = j * 16
            pos = o + lanes
            c0 = plsc.load_gather(c01_v, [2 * pos])
            c1 = plsc.load_gather(c01_v, [2 * pos + 1])
            cell = cell_v[pl.ds(o, 16)]
            reg = reg_v[pl.ds(o, 16)]
            clus = plsc.load_gather(labels_v, [cell])
            b0 = c0 // BINSIZE
            b1 = c1 // BINSIZE
            flat = reg * (N_CLUSTERS * PADW) + clus * PADW + b0
            idx_v[pl.ds(o, 16)] = flat
            lp1 = jnp.where(b0 == b1, cin, cout)
            plsc.store_scatter(out_v, [2 * pos + 1], lp1)
            return c

        lax.fori_loop(0, VECS, vec_body, 0)

        pltpu.async_copy(h_sp.at[idx_v], gath_v, sem).wait()

        def vec_body2(j, c):
            o = j * 16
            pos = o + lanes
            g = gath_v[pl.ds(o, 16)]
            plsc.store_scatter(out_v, [2 * pos], g)
            return c

        lax.fori_loop(0, VECS, vec_body2, 0)

        pltpu.sync_copy(out_v, out_hbm.at[pl.ds(2 * base, 2 * TILE)])
        return carry

    lax.fori_loop(0, n_tiles, tile_body, 0)


def kernel(baseline_weight, delta_logit_weight, inside, coordinates,
           local_region_ix, local_cell_ix, labels, regions_oi):
    heights = _heights(baseline_weight, delta_logit_weight, regions_oi)

    cflat = coordinates.reshape(2 * N_FRAG)
    sig = jax.nn.sigmoid(inside)
    c_in = jnp.log(sig) - math.log(BINWIDTH)
    c_out = jnp.log(1.0 - sig) - math.log(WIDTH - BINWIDTH)
    cin16 = jnp.broadcast_to(c_in, (16,)).astype(jnp.float32)
    cout16 = jnp.broadcast_to(c_out, (16,)).astype(jnp.float32)

    out = _frag_phase(heights.reshape(HEIGHTS_PAD), cflat,
                      local_region_ix, local_cell_ix, labels, cin16, cout16)
    return out.reshape(N_FRAG, 2)


# speedup vs baseline: 1.0558x; 1.0558x over previous
import functools
import math

import jax
import jax.numpy as jnp
from jax import lax
from jax.experimental import pallas as pl
from jax.experimental.pallas import tpu as pltpu
from jax.experimental.pallas import tpu_sc as plsc

BINSIZE = 200
WIDTH = 20000
BINWIDTH = 100
PADW = 128
N_REGIONS = 50000
N_CLUSTERS = 8
N_FRAG = 1000000
N_CELLS = 4096
N_REGIONS_OI = 1024

NC, NS = 2, 16
NW = NC * NS

TILE = 2000
NT = N_FRAG // TILE
VECS = TILE // 16

HEIGHTS_PAD = N_REGIONS_OI * N_CLUSTERS * PADW
HS = HEIGHTS_PAD // NS

_mesh = plsc.VectorSubcoreMesh(core_axis_name="c", subcore_axis_name="s")


_G = 16


def _heights_body(s_ref, *refs):
    in_refs = refs[:2 * _G]
    o_ref = refs[2 * _G]
    for g in range(_G):
        b = in_refs[g][...]
        d = in_refs[_G + g][...]
        u = b + d
        m = jnp.max(u, axis=-1, keepdims=True)
        lse = jnp.log(jnp.sum(jnp.exp(u - m), axis=-1, keepdims=True)) + m
        h = u - lse - math.log(BINSIZE)
        o_ref[g, :, :BINWIDTH] = h[0]


def _heights(baseline_weight, delta_logit_weight, regions_oi):
    def b_spec(g):
        return pl.BlockSpec((1, 1, BINWIDTH),
                            lambda i, s, g=g: (s[_G * i + g], 0, 0))

    def d_spec(g):
        return pl.BlockSpec((1, N_CLUSTERS, BINWIDTH),
                            lambda i, s, g=g: (s[_G * i + g], 0, 0))

    grid_spec = pltpu.PrefetchScalarGridSpec(
        num_scalar_prefetch=1,
        grid=(N_REGIONS_OI // _G,),
        in_specs=[b_spec(g) for g in range(_G)]
                 + [d_spec(g) for g in range(_G)],
        out_specs=pl.BlockSpec((_G, N_CLUSTERS, PADW),
                               lambda i, s: (i, 0, 0)),
    )
    return pl.pallas_call(
        _heights_body,
        grid_spec=grid_spec,
        out_shape=jax.ShapeDtypeStruct((N_REGIONS_OI, N_CLUSTERS, PADW),
                                       jnp.float32),
    )(regions_oi, *([baseline_weight.reshape(N_REGIONS, 1, BINWIDTH)] * _G),
      *([delta_logit_weight] * _G))


_TB = 16384
_NSTEP = (N_FRAG + _TB - 1) // _TB


def _pre_body(coord_ref, reg_ref, cin_ref, cout_ref, pidx_ref, lp1_ref):
    c = coord_ref[...]
    b0 = c[:, 0] // BINSIZE
    b1 = c[:, 1] // BINSIZE
    reg = reg_ref[...]
    pidx_ref[...] = reg * (N_CLUSTERS * PADW) + b0
    lp1_ref[...] = jnp.where(b0 == b1, cin_ref[0], cout_ref[0])


def _pre(coordinates, local_region_ix, cin, cout):
    return pl.pallas_call(
        _pre_body,
        grid=(_NSTEP,),
        in_specs=[
            pl.BlockSpec((_TB, 2), lambda i: (i, 0)),
            pl.BlockSpec((_TB,), lambda i: (i,)),
            pl.BlockSpec(memory_space=pltpu.SMEM),
            pl.BlockSpec(memory_space=pltpu.SMEM),
        ],
        out_specs=[
            pl.BlockSpec((_TB,), lambda i: (i,)),
            pl.BlockSpec((_TB,), lambda i: (i,)),
        ],
        out_shape=[
            jax.ShapeDtypeStruct((N_FRAG,), jnp.int32),
            jax.ShapeDtypeStruct((N_FRAG,), jnp.float32),
        ],
    )(coordinates, local_region_ix, cin, cout)


@functools.partial(
    pl.kernel,
    out_type=jax.ShapeDtypeStruct((N_FRAG,), jnp.float32),
    mesh=_mesh,
    compiler_params=pltpu.CompilerParams(needs_layout_passes=False),
    scratch_types=[
        pltpu.VMEM((N_CELLS,), jnp.int32),
        pltpu.VMEM((TILE,), jnp.int32),
        pltpu.VMEM((TILE,), jnp.int32),
        pltpu.VMEM((TILE,), jnp.int32),
        pltpu.VMEM((TILE,), jnp.float32),
        pltpu.VMEM_SHARED((HEIGHTS_PAD,), jnp.float32),
        pltpu.SemaphoreType.DMA,
    ],
)
def _frag_phase(h_hbm, pidx_hbm, cell_hbm, labels_hbm, out_hbm,
                labels_v, pidx_v, cell_v, idx_v, gath_v, h_sp, sem):
    sid = lax.axis_index("s")
    cid = lax.axis_index("c")
    wid = sid * NC + cid

    pltpu.sync_copy(h_hbm.at[pl.ds(sid * HS, HS)], h_sp.at[pl.ds(sid * HS, HS)])
    pltpu.sync_copy(labels_hbm, labels_v)
    plsc.subcore_barrier()

    n_tiles = (NT - wid + NW - 1) // NW

    def tile_body(i, carry):
        t = wid + i * NW
        base = t * TILE
        pltpu.sync_copy(pidx_hbm.at[pl.ds(base, TILE)], pidx_v)
        pltpu.sync_copy(cell_hbm.at[pl.ds(base, TILE)], cell_v)

        def vec_body(j, c):
            o = j * 16
            cell = cell_v[pl.ds(o, 16)]
            pidx = pidx_v[pl.ds(o, 16)]
            clus = plsc.load_gather(labels_v, [cell])
            idx_v[pl.ds(o, 16)] = pidx + clus * PADW
            return c

        lax.fori_loop(0, VECS, vec_body, 0)

        pltpu.async_copy(h_sp.at[idx_v], gath_v, sem).wait()
        pltpu.sync_copy(gath_v, out_hbm.at[pl.ds(base, TILE)])
        return carry

    lax.fori_loop(0, n_tiles, tile_body, 0)


def _merge_body(lp0_ref, lp1_ref, o_ref):
    o_ref[...] = jnp.concatenate(
        [lp0_ref[...][:, None], lp1_ref[...][:, None]], axis=1)


def _merge(lp0, lp1):
    return pl.pallas_call(
        _merge_body,
        grid=(_NSTEP,),
        in_specs=[
            pl.BlockSpec((_TB,), lambda i: (i,)),
            pl.BlockSpec((_TB,), lambda i: (i,)),
        ],
        out_specs=pl.BlockSpec((_TB, 2), lambda i: (i, 0)),
        out_shape=jax.ShapeDtypeStruct((N_FRAG, 2), jnp.float32),
    )(lp0, lp1)


def kernel(baseline_weight, delta_logit_weight, inside, coordinates,
           local_region_ix, local_cell_ix, labels, regions_oi):
    heights = _heights(baseline_weight, delta_logit_weight, regions_oi)

    sig = jax.nn.sigmoid(inside)
    c_in = (jnp.log(sig) - math.log(BINWIDTH)).astype(jnp.float32)
    c_out = (jnp.log(1.0 - sig) - math.log(WIDTH - BINWIDTH)).astype(jnp.float32)

    pidx, lp1 = _pre(coordinates, local_region_ix, c_in, c_out)
    lp0 = _frag_phase(heights.reshape(HEIGHTS_PAD), pidx,
                      local_cell_ix, labels)
    return _merge(lp0, lp1)


# speedup vs baseline: 5.5773x; 5.2824x over previous
import functools
import math

import jax
import jax.numpy as jnp
from jax import lax
from jax.experimental import pallas as pl
from jax.experimental.pallas import tpu as pltpu
from jax.experimental.pallas import tpu_sc as plsc

BINSIZE = 200
WIDTH = 20000
BINWIDTH = 100
PADW = 128
N_REGIONS = 50000
N_CLUSTERS = 8
N_FRAG = 1000000
N_CELLS = 4096
N_REGIONS_OI = 1024

NC, NS = 2, 16
NW = NC * NS

TILE = 2000
NT = N_FRAG // TILE
VECS = TILE // 16

HEIGHTS_PAD = N_REGIONS_OI * N_CLUSTERS * PADW
HS = HEIGHTS_PAD // NS

_mesh = plsc.VectorSubcoreMesh(core_axis_name="c", subcore_axis_name="s")


_G = 16


def _heights_body(s_ref, *refs):
    in_refs = refs[:2 * _G]
    o_ref = refs[2 * _G]
    for g in range(_G):
        b = in_refs[g][...]
        d = in_refs[_G + g][...]
        u = b + d
        m = jnp.max(u, axis=-1, keepdims=True)
        lse = jnp.log(jnp.sum(jnp.exp(u - m), axis=-1, keepdims=True)) + m
        h = u - lse - math.log(BINSIZE)
        o_ref[g, :, :BINWIDTH] = h[0]


def _heights(baseline_weight, delta_logit_weight, regions_oi):
    def b_spec(g):
        return pl.BlockSpec((1, 1, BINWIDTH),
                            lambda i, s, g=g: (s[_G * i + g], 0, 0))

    def d_spec(g):
        return pl.BlockSpec((1, N_CLUSTERS, BINWIDTH),
                            lambda i, s, g=g: (s[_G * i + g], 0, 0))

    grid_spec = pltpu.PrefetchScalarGridSpec(
        num_scalar_prefetch=1,
        grid=(N_REGIONS_OI // _G,),
        in_specs=[b_spec(g) for g in range(_G)]
                 + [d_spec(g) for g in range(_G)],
        out_specs=pl.BlockSpec((_G, N_CLUSTERS, PADW),
                               lambda i, s: (i, 0, 0)),
    )
    return pl.pallas_call(
        _heights_body,
        grid_spec=grid_spec,
        out_shape=jax.ShapeDtypeStruct((N_REGIONS_OI, N_CLUSTERS, PADW),
                                       jnp.float32),
    )(regions_oi, *([baseline_weight.reshape(N_REGIONS, 1, BINWIDTH)] * _G),
      *([delta_logit_weight] * _G))


@functools.partial(
    pl.kernel,
    out_type=jax.ShapeDtypeStruct((N_FRAG,), jnp.float32),
    mesh=_mesh,
    compiler_params=pltpu.CompilerParams(needs_layout_passes=False),
    scratch_types=[
        pltpu.VMEM((N_CELLS,), jnp.int32),
        pltpu.VMEM((TILE,), jnp.int32),
        pltpu.VMEM((TILE,), jnp.int32),
        pltpu.VMEM((TILE,), jnp.int32),
        pltpu.VMEM((TILE,), jnp.float32),
        pltpu.VMEM_SHARED((HEIGHTS_PAD,), jnp.float32),
        pltpu.SemaphoreType.DMA,
    ],
)
def _frag_phase(h_hbm, pidx_hbm, cell_hbm, labels_hbm, out_hbm,
                labels_v, pidx_v, cell_v, idx_v, gath_v, h_sp, sem):
    sid = lax.axis_index("s")
    cid = lax.axis_index("c")
    wid = sid * NC + cid

    pltpu.sync_copy(h_hbm.at[pl.ds(sid * HS, HS)], h_sp.at[pl.ds(sid * HS, HS)])
    pltpu.sync_copy(labels_hbm, labels_v)
    plsc.subcore_barrier()

    n_tiles = (NT - wid + NW - 1) // NW

    def tile_body(i, carry):
        t = wid + i * NW
        base = t * TILE
        pltpu.sync_copy(pidx_hbm.at[pl.ds(base, TILE)], pidx_v)
        pltpu.sync_copy(cell_hbm.at[pl.ds(base, TILE)], cell_v)

        def vec_body(j, c):
            o = j * 16
            cell = cell_v[pl.ds(o, 16)]
            pidx = pidx_v[pl.ds(o, 16)]
            clus = plsc.load_gather(labels_v, [cell])
            idx_v[pl.ds(o, 16)] = pidx + clus * PADW
            return c

        lax.fori_loop(0, VECS, vec_body, 0)

        pltpu.async_copy(h_sp.at[idx_v], gath_v, sem).wait()
        pltpu.sync_copy(gath_v, out_hbm.at[pl.ds(base, TILE)])
        return carry

    lax.fori_loop(0, n_tiles, tile_body, 0)


def kernel(baseline_weight, delta_logit_weight, inside, coordinates,
           local_region_ix, local_cell_ix, labels, regions_oi):
    heights = _heights(baseline_weight, delta_logit_weight, regions_oi)

    sig = jax.nn.sigmoid(inside)
    c_in = jnp.log(sig) - math.log(BINWIDTH)
    c_out = jnp.log(1.0 - sig) - math.log(WIDTH - BINWIDTH)

    b0 = coordinates[:, 0] // BINSIZE
    b1 = coordinates[:, 1] // BINSIZE
    pidx = local_region_ix * (N_CLUSTERS * PADW) + b0
    lp1 = jnp.where(b0 == b1, c_in, c_out).astype(jnp.float32)

    lp0 = _frag_phase(heights.reshape(HEIGHTS_PAD), pidx,
                      local_cell_ix, labels)
    return jnp.stack([lp0, lp1], axis=1)
